# baseline (speedup 1.0000x reference)
import jax
import jax.numpy as jnp
from jax.experimental import pallas as pl
from jax.experimental.pallas import tpu as pltpu

_BLOCK_P = 128


def _contract_kernel(x_ref, e_ref, o_ref):
    e = e_ref[...]
    for p in range(x_ref.shape[0]):
        x = x_ref[p].astype(jnp.bfloat16)
        o_ref[p] = jax.lax.dot_general(
            e,
            x,
            dimension_numbers=(((1,), (0,)), ((), ())),
            preferred_element_type=jnp.float32,
        )


def kernel(inputs, channel_embeddings):
    B, H, W, C = inputs.shape
    D = channel_embeddings.shape[1]
    P = B * H

    x_t = jnp.transpose(inputs, (0, 1, 3, 2)).reshape(P, C, W)
    e_t = jnp.transpose(channel_embeddings, (1, 0)).astype(jnp.bfloat16)

    out_t = pl.pallas_call(
        _contract_kernel,
        grid=(P // _BLOCK_P,),
        in_specs=[
            pl.BlockSpec((_BLOCK_P, C, W), lambda i: (i, 0, 0)),
            pl.BlockSpec((D, C), lambda i: (0, 0)),
        ],
        out_specs=pl.BlockSpec((_BLOCK_P, D, W), lambda i: (i, 0, 0)),
        out_shape=jax.ShapeDtypeStruct((P, D, W), jnp.float32),
        compiler_params=pltpu.CompilerParams(
            dimension_semantics=("arbitrary",),
        ),
    )(x_t, e_t)
    return jnp.transpose(out_t.reshape(B, H, D, W), (0, 1, 3, 2))
